# baseline (speedup 1.0000x reference)
import jax
import jax.numpy as jnp
from jax.experimental import pallas as pl
from jax.experimental.pallas import tpu as pltpu


def _copy_dma(in_ref, out_ref, sem):
    pltpu.make_async_copy(in_ref, out_ref, sem).start()
    pltpu.make_async_copy(in_ref, out_ref, sem).wait()


def kernel(x, emb_weight):
    sl = x.shape[1]
    model_dim = emb_weight.shape[1]
    return pl.pallas_call(
        _copy_dma,
        in_specs=[pl.BlockSpec(memory_space=pl.ANY)],
        out_specs=pl.BlockSpec(memory_space=pl.ANY),
        scratch_shapes=[pltpu.SemaphoreType.DMA],
        out_shape=jax.ShapeDtypeStruct((sl, model_dim), emb_weight.dtype),
    )(emb_weight[:sl])


# speedup vs baseline: 41.2344x; 41.2344x over previous
import jax
import jax.numpy as jnp
from jax.experimental import pallas as pl


def _copy_block(in_ref, out_ref):
    out_ref[...] = in_ref[...]


def kernel(x, emb_weight):
    sl = x.shape[1]
    model_dim = emb_weight.shape[1]
    block_rows = 512
    num_blocks = sl // block_rows
    return pl.pallas_call(
        _copy_block,
        grid=(num_blocks,),
        in_specs=[pl.BlockSpec((block_rows, model_dim), lambda i: (i, 0))],
        out_specs=pl.BlockSpec((block_rows, model_dim), lambda i: (i, 0)),
        out_shape=jax.ShapeDtypeStruct((sl, model_dim), emb_weight.dtype),
    )(emb_weight)


# speedup vs baseline: 48.9604x; 1.1874x over previous
import jax
import jax.numpy as jnp
from jax.experimental import pallas as pl


def _copy_block(in_ref, out_ref):
    out_ref[...] = in_ref[...]


def kernel(x, emb_weight):
    sl = x.shape[1]
    model_dim = emb_weight.shape[1]
    block_rows = 2048
    num_blocks = sl // block_rows
    return pl.pallas_call(
        _copy_block,
        grid=(num_blocks,),
        in_specs=[pl.BlockSpec((block_rows, model_dim), lambda i: (i, 0))],
        out_specs=pl.BlockSpec((block_rows, model_dim), lambda i: (i, 0)),
        out_shape=jax.ShapeDtypeStruct((sl, model_dim), emb_weight.dtype),
    )(emb_weight)
